# baseline (speedup 1.0000x reference)
import jax
import jax.numpy as jnp
from jax import lax
from jax.experimental import pallas as pl
from jax.experimental.pallas import tpu as pltpu

_C = 5
_CC = 6
_H = 384
_W = 384
_NH = 16
_NW = 16
_CH = 24
_CW = 24
_NSP = _NH * _NW
_NIT = 5
_BB = 2
_OFFS = tuple((dy, dx) for dy in (-1, 0, 1) for dx in (-1, 0, 1))
_NEG = -1e16
_ILN2 = 1.4426950408889634


def _vshift(a, dy):
    z_row = jnp.zeros((1, a.shape[1]), jnp.float32)
    if dy == 1:
        return jnp.concatenate([z_row, a[:-1, :]], axis=0)
    if dy == -1:
        return jnp.concatenate([a[1:, :], z_row], axis=0)
    return a


def _roll(a, s, ax):
    return pltpu.roll(a, s % a.shape[ax], ax)


def _rowsum(img):
    r = img.reshape(_NH, _CH, _W)
    return jnp.sum(r[:, 0:8, :] + r[:, 8:16, :] + r[:, 16:24, :], axis=1)


def _ssn_body(x_ref, q_ref, spf_ref, upad_ref):
    f32 = jnp.float32
    row = lax.broadcasted_iota(jnp.int32, (_NH, _H), 1) // _CH
    sub = lax.broadcasted_iota(jnp.int32, (_NH, _H), 0)
    P = (row == sub).astype(f32)
    Pt = P.T

    zs = lax.broadcasted_iota(jnp.int32, (_H, _W), 1)
    masks = {-1: zs < _W - _CW, 1: zs >= _CW}

    x = [[x_ref[bb, c] for c in range(_C)] for bb in range(_BB)]
    xs = [{
        -1: [_roll(x[bb][c], -_CW, 1) for c in range(_C)],
        0: x[bb],
        1: [_roll(x[bb][c], _CW, 1) for c in range(_C)],
    } for bb in range(_BB)]

    upad_ref[...] = jnp.zeros_like(upad_ref)
    neg_pad = jnp.full((_CH, _W), _NEG, f32)
    for bb in range(_BB):
        upad_ref[bb, 0:_CH, 5 * _W:6 * _W] = neg_pad
        upad_ref[bb, _CH + _H:, 5 * _W:6 * _W] = neg_pad

    G = []
    for bb in range(_BB):
        rs0 = jnp.concatenate([_rowsum(x[bb][c]) for c in range(_C)], axis=0)
        cs0 = jnp.dot(rs0, Pt,
                      preferred_element_type=f32) * f32(1.0 / (_CH * _CW))
        G.append([cs0[16 * c:16 * c + 16, :] for c in range(_C)])

    for it in range(_NIT):
        for bb in range(_BB):
            Gb = G[bb]
            nrm = Gb[0] * Gb[0]
            for c in range(1, _C):
                nrm = nrm + Gb[c] * Gb[c]
            gcat = jnp.concatenate(
                [Gb[c] * f32(2.0 * _ILN2) for c in range(_C)]
                + [nrm * f32(-_ILN2)], axis=0)
            s1 = jnp.dot(gcat, P, preferred_element_type=f32)
            for c in range(_CC):
                blk = s1[16 * c:16 * c + 16, :]
                rep = jnp.broadcast_to(blk[:, None, :],
                                       (_NH, _CH, _W)).reshape(_H, _W)
                upad_ref[bb, _CH:_CH + _H, c * _W:(c + 1) * _W] = rep

        eW = [None] * _BB
        rr = [None] * _BB
        rinv = [None] * _BB
        for bb in range(_BB):
            V = {}
            for dy in (-1, 0, 1):
                r0 = _CH + dy * _CH
                V[dy] = [upad_ref[bb, r0:r0 + _H, c * _W:(c + 1) * _W]
                         for c in range(_CC)]
            ew = {}
            for dy, dx in _OFFS:
                w = V[dy][_C] + xs[bb][dx][0] * V[dy][0]
                for c in range(1, _C):
                    w = w + xs[bb][dx][c] * V[dy][c]
                ew[(dy, dx)] = jnp.exp2(w if dx == 0 else
                                        jnp.where(masks[dx], w, f32(_NEG)))
            S = {dx: ew[(-1, dx)] + ew[(0, dx)] + ew[(1, dx)]
                 for dx in (-1, 0, 1)}
            s = S[0] + _roll(S[-1], _CW, 1) + _roll(S[1], -_CW, 1)
            ri = f32(1.0) / s
            if it == _NIT - 1:
                for k, (dy, dx) in enumerate(_OFFS):
                    e = ew[(dy, dx)]
                    q_ref[bb, k] = ri * (e if dx == 0 else
                                         _roll(e, -_CW * dx, 1))
            eW[bb] = ew
            rinv[bb] = ri
            rr[bb] = {
                -1: _roll(ri, -_CW, 1),
                0: ri,
                1: _roll(ri, _CW, 1),
            }

        for bb in range(_BB):
            pre = [None] * _CC
            for dy in (-1, 0, 1):
                t = {dx: eW[bb][(dy, dx)] * rr[bb][dx] for dx in (-1, 0, 1)}
                for c in range(_C):
                    y = (t[-1] * xs[bb][-1][c] + t[0] * xs[bb][0][c]
                         + t[1] * xs[bb][1][c])
                    rs = _vshift(_rowsum(y), dy)
                    pre[c] = rs if pre[c] is None else pre[c] + rs
                rs = _vshift(_rowsum(t[-1] + t[0] + t[1]), dy)
                pre[_C] = rs if pre[_C] is None else pre[_C] + rs
            cs = jnp.dot(jnp.concatenate(pre, axis=0), Pt,
                         preferred_element_type=f32)
            deni = f32(1.0) / (cs[16 * _C:16 * _C + 16, :] + f32(1e-16))
            G[bb] = [cs[16 * c:16 * c + 16, :] * deni for c in range(_C)]

    for bb in range(_BB):
        for i in range(_NH):
            blk = jnp.concatenate(
                [G[bb][c][i:i + 1, :] for c in range(_C)], axis=0)
            spf_ref[bb, :, i * _NW:(i + 1) * _NW] = blk


def kernel(x):
    b = x.shape[0]
    q, spf = pl.pallas_call(
        _ssn_body,
        grid=(b // _BB,),
        in_specs=[pl.BlockSpec((_BB, _C, _H, _W), lambda i: (i, 0, 0, 0))],
        out_specs=(
            pl.BlockSpec((_BB, 9, _H, _W), lambda i: (i, 0, 0, 0)),
            pl.BlockSpec((_BB, _C, _NSP), lambda i: (i, 0, 0)),
        ),
        out_shape=(
            jax.ShapeDtypeStruct((b, 9, _H, _W), jnp.float32),
            jax.ShapeDtypeStruct((b, _C, _NSP), jnp.float32),
        ),
        scratch_shapes=[
            pltpu.VMEM((_BB, _H + 2 * _CH, _CC * _W), jnp.float32),
        ],
        compiler_params=pltpu.CompilerParams(
            dimension_semantics=("parallel",)),
    )(x)
    return (q, x, spf, x)


# speedup vs baseline: 1.2386x; 1.2386x over previous
import jax
import jax.numpy as jnp
from jax import lax
from jax.experimental import pallas as pl
from jax.experimental.pallas import tpu as pltpu

_C = 5
_CC = 6
_H = 384
_W = 384
_NH = 16
_NW = 16
_CH = 24
_CW = 24
_NSP = _NH * _NW
_NIT = 5
_OFFS = tuple((dy, dx) for dy in (-1, 0, 1) for dx in (-1, 0, 1))
_NEG = -1e16
_ILN2 = 1.4426950408889634


def _vshift(a, dy):
    z_row = jnp.zeros((1, a.shape[1]), jnp.float32)
    if dy == 1:
        return jnp.concatenate([z_row, a[:-1, :]], axis=0)
    if dy == -1:
        return jnp.concatenate([a[1:, :], z_row], axis=0)
    return a


def _roll(a, s, ax):
    return pltpu.roll(a, s % a.shape[ax], ax)


def _rowsum(img):
    r = img.reshape(_NH, _CH, _W)
    return jnp.sum(r[:, 0:8, :] + r[:, 8:16, :] + r[:, 16:24, :], axis=1)


def _ssn_body(x_ref, q_ref, spf_ref, upad_ref):
    f32 = jnp.float32
    row = lax.broadcasted_iota(jnp.int32, (_NH, _H), 1) // _CH
    sub = lax.broadcasted_iota(jnp.int32, (_NH, _H), 0)
    P = (row == sub).astype(f32)
    Pt = P.T

    x = [x_ref[0, c] for c in range(_C)]
    xs = {
        -1: [_roll(x[c], -_CW, 1) for c in range(_C)],
        0: x,
        1: [_roll(x[c], _CW, 1) for c in range(_C)],
    }

    zs = lax.broadcasted_iota(jnp.int32, (_H, _W), 1)
    masks = {-1: zs < _W - _CW, 1: zs >= _CW}

    upad_ref[...] = jnp.zeros_like(upad_ref)
    neg_pad = jnp.full((_CH, _W), _NEG, f32)
    upad_ref[0:_CH, 5 * _W:6 * _W] = neg_pad
    upad_ref[_CH + _H:, 5 * _W:6 * _W] = neg_pad

    rs0 = jnp.concatenate([_rowsum(x[c]) for c in range(_C)], axis=0)
    cs0 = jnp.dot(rs0, Pt, preferred_element_type=f32) * f32(1.0 / (_CH * _CW))
    G = [cs0[16 * c:16 * c + 16, :] for c in range(_C)]

    for it in range(_NIT):
        nrm = G[0] * G[0]
        for c in range(1, _C):
            nrm = nrm + G[c] * G[c]
        gcat = jnp.concatenate(
            [G[c] * f32(2.0 * _ILN2) for c in range(_C)]
            + [nrm * f32(-_ILN2)], axis=0)
        s1 = jnp.dot(gcat, P, preferred_element_type=f32)
        reps = []
        for c in range(_CC):
            blk = s1[16 * c:16 * c + 16, :]
            rep = jnp.broadcast_to(blk[:, None, :],
                                   (_NH, _CH, _W)).reshape(_H, _W)
            upad_ref[_CH:_CH + _H, c * _W:(c + 1) * _W] = rep
            reps.append(rep)

        V = {0: reps}
        for dy in (-1, 1):
            r0 = _CH + dy * _CH
            V[dy] = [upad_ref[r0:r0 + _H, c * _W:(c + 1) * _W]
                     for c in range(_CC)]
        eW = {}
        for dy, dx in _OFFS:
            w = V[dy][_C] + xs[dx][0] * V[dy][0]
            for c in range(1, _C):
                w = w + xs[dx][c] * V[dy][c]
            eW[(dy, dx)] = jnp.exp2(w if dx == 0 else
                                    jnp.where(masks[dx], w, f32(_NEG)))
        S = {dx: eW[(-1, dx)] + eW[(0, dx)] + eW[(1, dx)]
             for dx in (-1, 0, 1)}
        s = S[0] + _roll(S[-1], _CW, 1) + _roll(S[1], -_CW, 1)
        rinv = f32(1.0) / s
        if it == _NIT - 1:
            for k, (dy, dx) in enumerate(_OFFS):
                e = eW[(dy, dx)]
                q_ref[0, k] = rinv * (e if dx == 0 else
                                      _roll(e, -_CW * dx, 1))
        rr = {
            -1: _roll(rinv, -_CW, 1),
            0: rinv,
            1: _roll(rinv, _CW, 1),
        }

        pre = [None] * _CC
        for dy in (-1, 0, 1):
            t = {dx: eW[(dy, dx)] * rr[dx] for dx in (-1, 0, 1)}
            for c in range(_C):
                y = t[-1] * xs[-1][c] + t[0] * xs[0][c] + t[1] * xs[1][c]
                rs = _vshift(_rowsum(y), dy)
                pre[c] = rs if pre[c] is None else pre[c] + rs
            rs = _vshift(_rowsum(t[-1] + t[0] + t[1]), dy)
            pre[_C] = rs if pre[_C] is None else pre[_C] + rs
        cs = jnp.dot(jnp.concatenate(pre, axis=0), Pt,
                     preferred_element_type=f32)
        deni = f32(1.0) / (cs[16 * _C:16 * _C + 16, :] + f32(1e-16))
        G = [cs[16 * c:16 * c + 16, :] * deni for c in range(_C)]

    for i in range(_NH):
        blk = jnp.concatenate([G[c][i:i + 1, :] for c in range(_C)], axis=0)
        spf_ref[0, :, i * _NW:(i + 1) * _NW] = blk


def kernel(x):
    b = x.shape[0]
    q, spf = pl.pallas_call(
        _ssn_body,
        grid=(b,),
        in_specs=[pl.BlockSpec((1, _C, _H, _W), lambda i: (i, 0, 0, 0))],
        out_specs=(
            pl.BlockSpec((1, 9, _H, _W), lambda i: (i, 0, 0, 0)),
            pl.BlockSpec((1, _C, _NSP), lambda i: (i, 0, 0)),
        ),
        out_shape=(
            jax.ShapeDtypeStruct((b, 9, _H, _W), jnp.float32),
            jax.ShapeDtypeStruct((b, _C, _NSP), jnp.float32),
        ),
        scratch_shapes=[
            pltpu.VMEM((_H + 2 * _CH, _CC * _W), jnp.float32),
        ],
        compiler_params=pltpu.CompilerParams(
            dimension_semantics=("parallel",)),
    )(x)
    return (q, x, spf, x)


# speedup vs baseline: 1.2392x; 1.0005x over previous
import jax
import jax.numpy as jnp
from jax import lax
from jax.experimental import pallas as pl
from jax.experimental.pallas import tpu as pltpu

_C = 5
_CC = 6
_H = 384
_W = 384
_NH = 16
_NW = 16
_CH = 24
_CW = 24
_NSP = _NH * _NW
_NIT = 5
_OFFS = tuple((dy, dx) for dy in (-1, 0, 1) for dx in (-1, 0, 1))
_NEG = -1e16
_ILN2 = 1.4426950408889634


def _vshift(a, dy):
    z_row = jnp.zeros((1, a.shape[1]), jnp.float32)
    if dy == 1:
        return jnp.concatenate([z_row, a[:-1, :]], axis=0)
    if dy == -1:
        return jnp.concatenate([a[1:, :], z_row], axis=0)
    return a


def _roll(a, s, ax):
    return pltpu.roll(a, s % a.shape[ax], ax)


def _rowsum(img):
    r = img.reshape(_NH, _CH, _W)
    return jnp.sum(r[:, 0:8, :] + r[:, 8:16, :] + r[:, 16:24, :], axis=1)


def _ssn_body(x_ref, q_ref, spf_ref, upad_ref):
    f32 = jnp.float32
    row = lax.broadcasted_iota(jnp.int32, (_NH, _H), 1) // _CH
    sub = lax.broadcasted_iota(jnp.int32, (_NH, _H), 0)
    P = (row == sub).astype(f32)
    Pt = P.T

    x = [x_ref[0, c] for c in range(_C)]
    xs = {
        -1: [_roll(x[c], -_CW, 1) for c in range(_C)],
        0: x,
        1: [_roll(x[c], _CW, 1) for c in range(_C)],
    }

    zs = lax.broadcasted_iota(jnp.int32, (_H, _W), 1)
    masks = {-1: zs < _W - _CW, 1: zs >= _CW}

    upad_ref[...] = jnp.zeros_like(upad_ref)
    neg_pad = jnp.full((_CH, _W), _NEG, f32)
    upad_ref[0:_CH, 5 * _W:6 * _W] = neg_pad
    upad_ref[_CH + _H:, 5 * _W:6 * _W] = neg_pad

    rs0 = jnp.concatenate([_rowsum(x[c]) for c in range(_C)], axis=0)
    cs0 = jnp.dot(rs0, Pt, preferred_element_type=f32) * f32(1.0 / (_CH * _CW))
    G = [cs0[16 * c:16 * c + 16, :] for c in range(_C)]

    for it in range(_NIT):
        nrm = G[0] * G[0]
        for c in range(1, _C):
            nrm = nrm + G[c] * G[c]
        gcat = jnp.concatenate(
            [G[c] * f32(2.0 * _ILN2) for c in range(_C)]
            + [nrm * f32(-_ILN2)], axis=0)
        s1 = jnp.dot(gcat, P, preferred_element_type=f32)
        for c in range(_CC):
            blk = s1[16 * c:16 * c + 16, :]
            rep = jnp.broadcast_to(blk[:, None, :],
                                   (_NH, _CH, _W)).reshape(_H, _W)
            upad_ref[_CH:_CH + _H, c * _W:(c + 1) * _W] = rep

        V = {}
        for dy in (-1, 0, 1):
            r0 = _CH + dy * _CH
            V[dy] = [upad_ref[r0:r0 + _H, c * _W:(c + 1) * _W]
                     for c in range(_CC)]
        eW = {}
        for dy, dx in _OFFS:
            w = V[dy][_C] + xs[dx][0] * V[dy][0]
            for c in range(1, _C):
                w = w + xs[dx][c] * V[dy][c]
            eW[(dy, dx)] = jnp.exp2(w if dx == 0 else
                                    jnp.where(masks[dx], w, f32(_NEG)))
        S = {dx: eW[(-1, dx)] + eW[(0, dx)] + eW[(1, dx)]
             for dx in (-1, 0, 1)}
        s = S[0] + _roll(S[-1], _CW, 1) + _roll(S[1], -_CW, 1)
        rinv = f32(1.0) / s
        if it == _NIT - 1:
            for k, (dy, dx) in enumerate(_OFFS):
                e = eW[(dy, dx)]
                q_ref[0, k] = rinv * (e if dx == 0 else
                                      _roll(e, -_CW * dx, 1))
        rr = {
            -1: _roll(rinv, -_CW, 1),
            0: rinv,
            1: _roll(rinv, _CW, 1),
        }

        pre = [None] * _CC
        for dy in (-1, 0, 1):
            t = {dx: eW[(dy, dx)] * rr[dx] for dx in (-1, 0, 1)}
            for c in range(_C):
                y = t[-1] * xs[-1][c] + t[0] * xs[0][c] + t[1] * xs[1][c]
                rs = _vshift(_rowsum(y), dy)
                pre[c] = rs if pre[c] is None else pre[c] + rs
            rs = _vshift(_rowsum(t[-1] + t[0] + t[1]), dy)
            pre[_C] = rs if pre[_C] is None else pre[_C] + rs
        cs = jnp.dot(jnp.concatenate(pre, axis=0), Pt,
                     preferred_element_type=f32)
        deni = f32(1.0) / (cs[16 * _C:16 * _C + 16, :] + f32(1e-16))
        G = [cs[16 * c:16 * c + 16, :] * deni for c in range(_C)]

    for i in range(_NH):
        blk = jnp.concatenate([G[c][i:i + 1, :] for c in range(_C)], axis=0)
        spf_ref[0, :, i * _NW:(i + 1) * _NW] = blk


def kernel(x):
    b = x.shape[0]
    q, spf = pl.pallas_call(
        _ssn_body,
        grid=(b,),
        in_specs=[pl.BlockSpec((1, _C, _H, _W), lambda i: (i, 0, 0, 0))],
        out_specs=(
            pl.BlockSpec((1, 9, _H, _W), lambda i: (i, 0, 0, 0)),
            pl.BlockSpec((1, _C, _NSP), lambda i: (i, 0, 0)),
        ),
        out_shape=(
            jax.ShapeDtypeStruct((b, 9, _H, _W), jnp.float32),
            jax.ShapeDtypeStruct((b, _C, _NSP), jnp.float32),
        ),
        scratch_shapes=[
            pltpu.VMEM((_H + 2 * _CH, _CC * _W), jnp.float32),
        ],
        compiler_params=pltpu.CompilerParams(
            dimension_semantics=("parallel",)),
    )(x)
    return (q, x, spf, x)
